# Initial kernel scaffold; baseline (speedup 1.0000x reference)
import jax
import jax.numpy as jnp
from jax.experimental import pallas as pl


def kernel(x, conv1_w, conv1_b, conv2_w, conv2_b, conv3_w, conv3_b, conv4_w, conv4_b, fc1_w, fc1_b, fc2_w, fc2_b, fc3_w, fc3_b):
    raise NotImplementedError("write your pallas kernel here")


# speedup vs baseline: 1.0653x; 1.0653x over previous
import functools

import jax
import jax.numpy as jnp
from jax.experimental import pallas as pl
from jax.experimental.pallas import tpu as pltpu


def _conv_stack_kernel(x_ref, w1, w2, w3, w4, b1, b2, b3, b4, o_ref,
                       pad1, patch1, acc1, pad2, patch2, acc2,
                       pad3, patch3, acc3, pad4, patch4, acc4):
    f32 = jnp.float32

    def zero_border(p):
        P, Q, C = p.shape
        p[0:1, :, :] = jnp.zeros((1, Q, C), p.dtype)
        p[P - 1:P, :, :] = jnp.zeros((1, Q, C), p.dtype)
        p[:, 0:1, :] = jnp.zeros((P, 1, C), p.dtype)
        p[:, Q - 1:Q, :] = jnp.zeros((P, 1, C), p.dtype)

    def im2col(pad, patch, H, W, cin):
        for dy in range(3):
            for dx in range(3):
                t = dy * 3 + dx
                patch[:, :, t * cin:(t + 1) * cin] = pad[dy:dy + H, dx:dx + W, :]

    def conv(patch, w, b, acc, H, W, cin):
        p2 = patch[...].reshape(H * W, 9 * cin)
        acc[...] = jnp.dot(p2, w[...], preferred_element_type=f32) + b[...]

    zero_border(pad1)
    zero_border(pad2)
    zero_border(pad3)
    zero_border(pad4)

    pad1[1:65, 1:65, :] = x_ref[0]
    im2col(pad1, patch1, 64, 64, 3)
    conv(patch1, w1, b1, acc1, 64, 64, 3)
    pad2[1:65, 1:65, :] = acc1[...].reshape(64, 64, 32).astype(jnp.bfloat16)

    im2col(pad2, patch2, 64, 64, 32)
    conv(patch2, w2, b2, acc2, 64, 64, 32)
    v = acc2[...].reshape(32, 2, 32, 2, 64)
    pooled = jnp.max(jnp.max(v, axis=3), axis=1)
    pad3[1:33, 1:33, :] = pooled.astype(jnp.bfloat16)

    im2col(pad3, patch3, 32, 32, 64)
    conv(patch3, w3, b3, acc3, 32, 32, 64)
    pad4[1:33, 1:33, :] = acc3[...].reshape(32, 32, 128).astype(jnp.bfloat16)

    im2col(pad4, patch4, 32, 32, 128)
    conv(patch4, w4, b4, acc4, 32, 32, 128)
    v = acc4[...].reshape(16, 2, 16, 2, 128)
    pooled = jnp.max(jnp.max(v, axis=3), axis=1)
    o_ref[0] = pooled.astype(o_ref.dtype)


def _conv_stack(x_nhwc, w1, b1, w2, b2, w3, b3, w4, b4):
    N = x_nhwc.shape[0]
    bf16 = jnp.bfloat16

    flops = 2 * N * (64 * 64 * 32 * 27 + 64 * 64 * 64 * 288
                     + 32 * 32 * 128 * 576 + 32 * 32 * 128 * 1152)
    bytes_accessed = N * (64 * 64 * 3 * 2 + 16 * 16 * 128 * 2)

    return pl.pallas_call(
        _conv_stack_kernel,
        out_shape=jax.ShapeDtypeStruct((N, 16, 16, 128), bf16),
        grid_spec=pltpu.PrefetchScalarGridSpec(
            num_scalar_prefetch=0,
            grid=(N,),
            in_specs=[
                pl.BlockSpec((1, 64, 64, 3), lambda n: (n, 0, 0, 0)),
                pl.BlockSpec((27, 32), lambda n: (0, 0)),
                pl.BlockSpec((288, 64), lambda n: (0, 0)),
                pl.BlockSpec((576, 128), lambda n: (0, 0)),
                pl.BlockSpec((1152, 128), lambda n: (0, 0)),
                pl.BlockSpec((1, 32), lambda n: (0, 0)),
                pl.BlockSpec((1, 64), lambda n: (0, 0)),
                pl.BlockSpec((1, 128), lambda n: (0, 0)),
                pl.BlockSpec((1, 128), lambda n: (0, 0)),
            ],
            out_specs=pl.BlockSpec((1, 16, 16, 128), lambda n: (n, 0, 0, 0)),
            scratch_shapes=[
                pltpu.VMEM((66, 66, 3), bf16),
                pltpu.VMEM((64, 64, 27), bf16),
                pltpu.VMEM((4096, 32), jnp.float32),
                pltpu.VMEM((66, 66, 32), bf16),
                pltpu.VMEM((64, 64, 288), bf16),
                pltpu.VMEM((4096, 64), jnp.float32),
                pltpu.VMEM((34, 34, 64), bf16),
                pltpu.VMEM((32, 32, 576), bf16),
                pltpu.VMEM((1024, 128), jnp.float32),
                pltpu.VMEM((34, 34, 128), bf16),
                pltpu.VMEM((32, 32, 1152), bf16),
                pltpu.VMEM((1024, 128), jnp.float32),
            ],
        ),
        compiler_params=pltpu.CompilerParams(
            dimension_semantics=("parallel",),
            vmem_limit_bytes=32 * 1024 * 1024),
        cost_estimate=pl.CostEstimate(flops=flops, transcendentals=0,
                                      bytes_accessed=bytes_accessed),
    )(x_nhwc, w1, w2, w3, w4,
      b1.reshape(1, 32), b2.reshape(1, 64),
      b3.reshape(1, 128), b4.reshape(1, 128))


def _fc1_kernel(x_ref, w_ref, b_ref, o_ref, acc_ref):
    k = pl.program_id(1)

    @pl.when(k == 0)
    def _init():
        acc_ref[...] = jnp.zeros_like(acc_ref)

    acc_ref[...] += jnp.dot(x_ref[...], w_ref[...],
                            preferred_element_type=jnp.float32)

    @pl.when(k == pl.num_programs(1) - 1)
    def _fin():
        o_ref[...] = jnp.maximum(acc_ref[...] + b_ref[...],
                                 0.0).astype(o_ref.dtype)


def _fc1(feat, w, b):
    M, K = feat.shape
    _, N = w.shape
    tn, tk = N // 2, 2048
    grid = (N // tn, K // tk)

    flops = 2 * M * K * N
    bytes_accessed = M * K * 2 + K * N * 2 + N * 4 + M * N * 2

    return pl.pallas_call(
        _fc1_kernel,
        out_shape=jax.ShapeDtypeStruct((M, N), jnp.bfloat16),
        grid_spec=pltpu.PrefetchScalarGridSpec(
            num_scalar_prefetch=0,
            grid=grid,
            in_specs=[
                pl.BlockSpec((M, tk), lambda j, k: (0, k)),
                pl.BlockSpec((tk, tn), lambda j, k: (k, j)),
                pl.BlockSpec((1, tn), lambda j, k: (0, j)),
            ],
            out_specs=pl.BlockSpec((M, tn), lambda j, k: (0, j)),
            scratch_shapes=[pltpu.VMEM((M, tn), jnp.float32)],
        ),
        compiler_params=pltpu.CompilerParams(
            dimension_semantics=("parallel", "arbitrary"),
            vmem_limit_bytes=32 * 1024 * 1024),
        cost_estimate=pl.CostEstimate(flops=flops, transcendentals=0,
                                      bytes_accessed=bytes_accessed),
    )(feat, w, b.reshape(1, N))


def _fc23_kernel(h_ref, w2_ref, b2_ref, w3_ref, b3_ref, o_ref):
    f32 = jnp.float32
    h2 = jnp.dot(h_ref[...], w2_ref[...], preferred_element_type=f32)
    h2 = jnp.maximum(h2 + b2_ref[...], 0.0).astype(jnp.bfloat16)
    o_ref[...] = jnp.dot(h2, w3_ref[...],
                         preferred_element_type=f32) + b3_ref[...]


def _fc23(h, w2, b2, w3p, b3p):
    M = h.shape[0]
    N2 = w2.shape[1]
    N3 = w3p.shape[1]
    return pl.pallas_call(
        _fc23_kernel,
        out_shape=jax.ShapeDtypeStruct((M, N3), jnp.float32),
        grid_spec=pltpu.PrefetchScalarGridSpec(
            num_scalar_prefetch=0,
            grid=(1,),
            in_specs=[
                pl.BlockSpec((M, 1024), lambda i: (0, 0)),
                pl.BlockSpec((1024, N2), lambda i: (0, 0)),
                pl.BlockSpec((1, N2), lambda i: (0, 0)),
                pl.BlockSpec((512, N3), lambda i: (0, 0)),
                pl.BlockSpec((1, N3), lambda i: (0, 0)),
            ],
            out_specs=pl.BlockSpec((M, N3), lambda i: (0, 0)),
            scratch_shapes=[],
        ),
        compiler_params=pltpu.CompilerParams(
            dimension_semantics=("arbitrary",),
            vmem_limit_bytes=16 * 1024 * 1024),
    )(h, w2, b2.reshape(1, N2), w3p, b3p.reshape(1, N3))


def kernel(x, conv1_w, conv1_b, conv2_w, conv2_b, conv3_w, conv3_b,
           conv4_w, conv4_b, fc1_w, fc1_b, fc2_w, fc2_b, fc3_w, fc3_b):
    bf16 = jnp.bfloat16
    x_nhwc = jnp.transpose(x, (0, 2, 3, 1)).astype(bf16)

    out = _conv_stack(x_nhwc,
                      conv1_w.astype(bf16), conv1_b.astype(jnp.float32),
                      conv2_w.astype(bf16), conv2_b.astype(jnp.float32),
                      conv3_w.astype(bf16), conv3_b.astype(jnp.float32),
                      conv4_w.astype(bf16), conv4_b.astype(jnp.float32))

    feat = out.reshape(out.shape[0], -1)
    h = _fc1(feat, fc1_w.astype(bf16), fc1_b.astype(jnp.float32))

    num_classes = fc3_w.shape[1]
    n3p = ((num_classes + 127) // 128) * 128
    w3p = jnp.pad(fc3_w.astype(bf16), ((0, 0), (0, n3p - num_classes)))
    b3p = jnp.pad(fc3_b.astype(jnp.float32), (0, n3p - num_classes))

    logits = _fc23(h, fc2_w.astype(bf16), fc2_b.astype(jnp.float32), w3p, b3p)
    return logits[:, :num_classes]


# speedup vs baseline: 4.0041x; 3.7586x over previous
import functools

import jax
import jax.numpy as jnp
from jax.experimental import pallas as pl
from jax.experimental.pallas import tpu as pltpu


def _pack_conv_weights(w, cin, cout, f):
    wr = w.reshape(3, 3, cin, cout)
    zero = jnp.zeros((3, cin, cout), w.dtype)
    rows = []
    for q in range(f + 2):
        cols = []
        for p in range(f):
            dx = q - p
            cols.append(wr[:, dx] if 0 <= dx <= 2 else zero)
        rows.append(jnp.concatenate(cols, axis=2))
    wp = jnp.concatenate(rows, axis=1)
    return wp.reshape(3 * (f + 2) * cin, f * cout)


def _conv_stack_kernel(x_ref, w1, w2, w3, w4, b1, b2, b3, b4, o_ref,
                       q1, acc1, q2, acc2, q3, acc3, q4, acc4):
    bf16 = jnp.bfloat16
    f32 = jnp.float32

    def conv3tap(q, w, b, H, Wq, K):
        return (
            jnp.dot(q[0:H].reshape(H * Wq, K), w[0:K],
                    preferred_element_type=f32)
            + jnp.dot(q[1:H + 1].reshape(H * Wq, K), w[K:2 * K],
                      preferred_element_type=f32)
            + jnp.dot(q[2:H + 2].reshape(H * Wq, K), w[2 * K:3 * K],
                      preferred_element_type=f32)
            + b[...])

    q1[:, :, 3:15] = x_ref[0]
    q1[:, 1:16, 0:3] = x_ref[0, :, 0:15, 9:12]
    q1[:, 0:1, 0:3] = jnp.zeros((66, 1, 3), bf16)
    q1[:, 0:15, 15:18] = x_ref[0, :, 1:16, 0:3]
    q1[:, 15:16, 15:18] = jnp.zeros((66, 1, 3), bf16)
    acc1[...] = conv3tap(q1, w1, b1, 64, 16, 18)

    v = acc1[...].astype(bf16).reshape(64, 16, 128)
    q2[0:1] = jnp.zeros((1, 16, 192), bf16)
    q2[65:66] = jnp.zeros((1, 16, 192), bf16)
    q2[:, 0:1, 0:32] = jnp.zeros((66, 1, 32), bf16)
    q2[:, 15:16, 160:192] = jnp.zeros((66, 1, 32), bf16)
    q2[1:65, 1:16, 0:32] = v[:, 0:15, 96:128]
    q2[1:65, :, 32:160] = v
    q2[1:65, 0:15, 160:192] = v[:, 1:16, 0:32]

    acc2[...] = conv3tap(q2, w2, b2, 64, 16, 192)
    a = acc2[...].reshape(64, 16, 256)
    p_lo = jnp.maximum(a[:, :, 0:64], a[:, :, 64:128])
    p_hi = jnp.maximum(a[:, :, 128:192], a[:, :, 192:256])
    v_lo = jnp.max(p_lo.reshape(32, 2, 16, 64), axis=1).astype(bf16)
    v_hi = jnp.max(p_hi.reshape(32, 2, 16, 64), axis=1).astype(bf16)

    q3[0:1] = jnp.zeros((1, 16, 256), bf16)
    q3[33:34] = jnp.zeros((1, 16, 256), bf16)
    q3[:, 0:1, 0:64] = jnp.zeros((34, 1, 64), bf16)
    q3[:, 15:16, 192:256] = jnp.zeros((34, 1, 64), bf16)
    q3[1:33, 1:16, 0:64] = v_hi[:, 0:15, :]
    q3[1:33, :, 64:128] = v_lo
    q3[1:33, :, 128:192] = v_hi
    q3[1:33, 0:15, 192:256] = v_lo[:, 1:16, :]

    acc3[...] = conv3tap(q3, w3, b3, 32, 16, 256)

    v = acc3[...].astype(bf16).reshape(32, 16, 256)
    q4[0:1] = jnp.zeros((1, 16, 512), bf16)
    q4[33:34] = jnp.zeros((1, 16, 512), bf16)
    q4[:, 0:1, 0:128] = jnp.zeros((34, 1, 128), bf16)
    q4[:, 15:16, 384:512] = jnp.zeros((34, 1, 128), bf16)
    q4[1:33, 1:16, 0:128] = v[:, 0:15, 128:256]
    q4[1:33, :, 128:384] = v
    q4[1:33, 0:15, 384:512] = v[:, 1:16, 0:128]

    acc4[...] = conv3tap(q4, w4, b4, 32, 16, 512)
    a = acc4[...].reshape(32, 16, 256)
    p = jnp.maximum(a[:, :, 0:128], a[:, :, 128:256])
    p = jnp.max(p.reshape(16, 2, 16, 128), axis=1)
    o_ref[0] = p.astype(o_ref.dtype)


def _conv_stack(x_pack, w1, w2, w3, w4, b1, b2, b3, b4):
    N = x_pack.shape[0]
    bf16 = jnp.bfloat16

    flops = 2 * N * (64 * 64 * 32 * 27 + 64 * 64 * 64 * 288
                     + 32 * 32 * 128 * 576 + 32 * 32 * 128 * 1152)
    bytes_accessed = N * (66 * 16 * 12 * 2 + 16 * 16 * 128 * 2)

    return pl.pallas_call(
        _conv_stack_kernel,
        out_shape=jax.ShapeDtypeStruct((N, 16, 16, 128), bf16),
        grid_spec=pltpu.PrefetchScalarGridSpec(
            num_scalar_prefetch=0,
            grid=(N,),
            in_specs=[
                pl.BlockSpec((1, 66, 16, 12), lambda n: (n, 0, 0, 0)),
                pl.BlockSpec((54, 128), lambda n: (0, 0)),
                pl.BlockSpec((576, 256), lambda n: (0, 0)),
                pl.BlockSpec((768, 256), lambda n: (0, 0)),
                pl.BlockSpec((1536, 256), lambda n: (0, 0)),
                pl.BlockSpec((1, 128), lambda n: (0, 0)),
                pl.BlockSpec((1, 256), lambda n: (0, 0)),
                pl.BlockSpec((1, 256), lambda n: (0, 0)),
                pl.BlockSpec((1, 256), lambda n: (0, 0)),
            ],
            out_specs=pl.BlockSpec((1, 16, 16, 128), lambda n: (n, 0, 0, 0)),
            scratch_shapes=[
                pltpu.VMEM((66, 16, 18), bf16),
                pltpu.VMEM((1024, 128), jnp.float32),
                pltpu.VMEM((66, 16, 192), bf16),
                pltpu.VMEM((1024, 256), jnp.float32),
                pltpu.VMEM((34, 16, 256), bf16),
                pltpu.VMEM((512, 256), jnp.float32),
                pltpu.VMEM((34, 16, 512), bf16),
                pltpu.VMEM((512, 256), jnp.float32),
            ],
        ),
        compiler_params=pltpu.CompilerParams(
            dimension_semantics=("parallel",),
            vmem_limit_bytes=32 * 1024 * 1024),
        cost_estimate=pl.CostEstimate(flops=flops, transcendentals=0,
                                      bytes_accessed=bytes_accessed),
    )(x_pack, w1, w2, w3, w4, b1, b2, b3, b4)


def _fc1_kernel(x_ref, w_ref, b_ref, o_ref, acc_ref):
    k = pl.program_id(1)

    @pl.when(k == 0)
    def _init():
        acc_ref[...] = jnp.zeros_like(acc_ref)

    acc_ref[...] += jnp.dot(x_ref[...], w_ref[...],
                            preferred_element_type=jnp.float32)

    @pl.when(k == pl.num_programs(1) - 1)
    def _fin():
        o_ref[...] = jnp.maximum(acc_ref[...] + b_ref[...],
                                 0.0).astype(o_ref.dtype)


def _fc1(feat, w, b):
    M, K = feat.shape
    _, N = w.shape
    tn, tk = N // 2, 2048
    grid = (N // tn, K // tk)

    flops = 2 * M * K * N
    bytes_accessed = M * K * 2 + K * N * 2 + N * 4 + M * N * 2

    return pl.pallas_call(
        _fc1_kernel,
        out_shape=jax.ShapeDtypeStruct((M, N), jnp.bfloat16),
        grid_spec=pltpu.PrefetchScalarGridSpec(
            num_scalar_prefetch=0,
            grid=grid,
            in_specs=[
                pl.BlockSpec((M, tk), lambda j, k: (0, k)),
                pl.BlockSpec((tk, tn), lambda j, k: (k, j)),
                pl.BlockSpec((1, tn), lambda j, k: (0, j)),
            ],
            out_specs=pl.BlockSpec((M, tn), lambda j, k: (0, j)),
            scratch_shapes=[pltpu.VMEM((M, tn), jnp.float32)],
        ),
        compiler_params=pltpu.CompilerParams(
            dimension_semantics=("parallel", "arbitrary"),
            vmem_limit_bytes=32 * 1024 * 1024),
        cost_estimate=pl.CostEstimate(flops=flops, transcendentals=0,
                                      bytes_accessed=bytes_accessed),
    )(feat, w, b.reshape(1, N))


def _fc23_kernel(h_ref, w2_ref, b2_ref, w3_ref, b3_ref, o_ref):
    f32 = jnp.float32
    h2 = jnp.dot(h_ref[...], w2_ref[...], preferred_element_type=f32)
    h2 = jnp.maximum(h2 + b2_ref[...], 0.0).astype(jnp.bfloat16)
    o_ref[...] = jnp.dot(h2, w3_ref[...],
                         preferred_element_type=f32) + b3_ref[...]


def _fc23(h, w2, b2, w3p, b3p):
    M = h.shape[0]
    N2 = w2.shape[1]
    N3 = w3p.shape[1]
    return pl.pallas_call(
        _fc23_kernel,
        out_shape=jax.ShapeDtypeStruct((M, N3), jnp.float32),
        grid_spec=pltpu.PrefetchScalarGridSpec(
            num_scalar_prefetch=0,
            grid=(1,),
            in_specs=[
                pl.BlockSpec((M, 1024), lambda i: (0, 0)),
                pl.BlockSpec((1024, N2), lambda i: (0, 0)),
                pl.BlockSpec((1, N2), lambda i: (0, 0)),
                pl.BlockSpec((512, N3), lambda i: (0, 0)),
                pl.BlockSpec((1, N3), lambda i: (0, 0)),
            ],
            out_specs=pl.BlockSpec((M, N3), lambda i: (0, 0)),
            scratch_shapes=[],
        ),
        compiler_params=pltpu.CompilerParams(
            dimension_semantics=("arbitrary",),
            vmem_limit_bytes=16 * 1024 * 1024),
    )(h, w2, b2.reshape(1, N2), w3p, b3p.reshape(1, N3))


def kernel(x, conv1_w, conv1_b, conv2_w, conv2_b, conv3_w, conv3_b,
           conv4_w, conv4_b, fc1_w, fc1_b, fc2_w, fc2_b, fc3_w, fc3_b):
    bf16 = jnp.bfloat16
    f32 = jnp.float32

    x_nhwc = jnp.transpose(x, (0, 2, 3, 1)).astype(bf16)
    x_pack = jnp.pad(x_nhwc.reshape(x.shape[0], 64, 16, 12),
                     ((0, 0), (1, 1), (0, 0), (0, 0)))

    w1p = _pack_conv_weights(conv1_w.astype(bf16), 3, 32, 4)
    w2p = _pack_conv_weights(conv2_w.astype(bf16), 32, 64, 4)
    w3p = _pack_conv_weights(conv3_w.astype(bf16), 64, 128, 2)
    w4p = _pack_conv_weights(conv4_w.astype(bf16), 128, 128, 2)
    b1p = jnp.tile(conv1_b.astype(f32), 4).reshape(1, 128)
    b2p = jnp.tile(conv2_b.astype(f32), 4).reshape(1, 256)
    b3p = jnp.tile(conv3_b.astype(f32), 2).reshape(1, 256)
    b4p = jnp.tile(conv4_b.astype(f32), 2).reshape(1, 256)

    out = _conv_stack(x_pack, w1p, w2p, w3p, w4p, b1p, b2p, b3p, b4p)

    feat = out.reshape(out.shape[0], -1)
    h = _fc1(feat, fc1_w.astype(bf16), fc1_b.astype(f32))

    num_classes = fc3_w.shape[1]
    n3p = ((num_classes + 127) // 128) * 128
    fw3p = jnp.pad(fc3_w.astype(bf16), ((0, 0), (0, n3p - num_classes)))
    fb3p = jnp.pad(fc3_b.astype(f32), (0, n3p - num_classes))

    logits = _fc23(h, fc2_w.astype(bf16), fc2_b.astype(f32), fw3p, fb3p)
    return logits[:, :num_classes]
